# baseline (speedup 1.0000x reference)
import functools

import jax
import jax.numpy as jnp
from jax import lax
from jax.experimental import pallas as pl
from jax.experimental.pallas import tpu as pltpu
from jax.experimental.pallas import tpu_sc as plsc

N_HID = 128
MAX_LEN = 240
LANES = 16
CHUNK = 128
NSLOT = 7


def _proj_table_body(emb_ref, w_ref, b_ref, out_ref):
    p = lax.dot_general(
        emb_ref[...], w_ref[...],
        dimension_numbers=(((1,), (1,)), ((), ())),
        preferred_element_type=jnp.float32,
    )
    out_ref[...] = p + b_ref[...]


@functools.cache
def _make_sc_kernel(n_rows):
    n_chunks = n_rows // CHUNK
    info = plsc.get_sparse_core_info()
    nc, ns = info.num_cores, info.num_subcores
    nw = nc * ns

    mesh = plsc.VectorSubcoreMesh(core_axis_name="c", subcore_axis_name="s")

    @functools.partial(
        pl.kernel,
        mesh=mesh,
        out_type=jax.ShapeDtypeStruct((n_rows, N_HID), jnp.float32),
        scratch_types=(
            [pltpu.VMEM((CHUNK,), jnp.int32) for _ in range(NSLOT)]
            + [pltpu.VMEM((CHUNK, N_HID), jnp.float32)
               for _ in range(NSLOT)]
            + [pltpu.VMEM_SHARED((MAX_LEN, N_HID), jnp.float32)]
            + [pltpu.SemaphoreType.DMA for _ in range(4 * NSLOT)]
        ),
    )
    def sc_fn(x_hbm, t_hbm, p_hbm, out_hbm, *scratch):
        iv = scratch[0:NSLOT]
        xv = scratch[NSLOT:2 * NSLOT]
        p_sh = scratch[2 * NSLOT]
        si = scratch[2 * NSLOT + 1:2 * NSLOT + 1 + NSLOT]
        sx = scratch[2 * NSLOT + 1 + NSLOT:2 * NSLOT + 1 + 2 * NSLOT]
        sg = scratch[2 * NSLOT + 1 + 2 * NSLOT:2 * NSLOT + 1 + 3 * NSLOT]
        so = scratch[2 * NSLOT + 1 + 3 * NSLOT:2 * NSLOT + 1 + 4 * NSLOT]

        wid = lax.axis_index("s") * nc + lax.axis_index("c")

        @pl.when(lax.axis_index("s") == 0)
        def _():
            pltpu.sync_copy(p_hbm, p_sh)
        plsc.subcore_barrier()

        def valid(m):
            return wid + m * nw < n_chunks

        def row_base(m):
            return (wid + m * nw) * CHUNK

        def issue_idx(m, s):
            @pl.when(valid(m))
            def _():
                pltpu.async_copy(
                    t_hbm.at[pl.ds(row_base(m), CHUNK)], iv[s], si[s])

        def issue_x(m, s):
            @pl.when(valid(m))
            def _():
                @pl.when(m >= NSLOT)
                def _():
                    pltpu.make_async_copy(
                        xv[s], out_hbm.at[pl.ds(0, CHUNK)], so[s]).wait()

                pltpu.async_copy(
                    x_hbm.at[pl.ds(row_base(m), CHUNK)], xv[s], sx[s])

        def issue_gadd(m, s):
            @pl.when(valid(m))
            def _():
                pltpu.make_async_copy(
                    t_hbm.at[pl.ds(row_base(m), CHUNK)], iv[s], si[s]).wait()
                pltpu.make_async_copy(
                    x_hbm.at[pl.ds(row_base(m), CHUNK)], xv[s], sx[s]).wait()
                pltpu.async_copy(p_sh.at[iv[s]], xv[s], sg[s], add=True)

        def finish(m, s):
            @pl.when(valid(m))
            def _():
                pltpu.make_async_copy(
                    p_sh.at[iv[s]], xv[s], sg[s]).wait()
                pltpu.async_copy(
                    xv[s], out_hbm.at[pl.ds(row_base(m), CHUNK)], so[s])

        for mm in range(NSLOT):
            issue_idx(mm, mm)
        issue_x(0, 0)
        issue_x(1, 1)
        issue_x(2, 2)
        issue_gadd(0, 0)

        def body(g, carry):
            for dm in range(NSLOT):
                m = g * NSLOT + dm
                issue_gadd(m + 1, (dm + 1) % NSLOT)
                finish(m, dm % NSLOT)
                issue_x(m + 3, (dm + 3) % NSLOT)
                issue_idx(m + NSLOT, dm % NSLOT)
            return carry

        per_w_max = -(-n_chunks // nw)
        lax.fori_loop(0, (per_w_max + NSLOT - 1) // NSLOT, body, 0)

        for s in range(NSLOT):
            pltpu.make_async_copy(
                xv[s], out_hbm.at[pl.ds(0, CHUNK)], so[s]).wait()

    return sc_fn


def kernel(x, t, emb_table, W, b):
    p_table = pl.pallas_call(
        _proj_table_body,
        out_shape=jax.ShapeDtypeStruct((MAX_LEN, N_HID), jnp.float32),
    )(emb_table, W, b.reshape(1, N_HID))
    return _make_sc_kernel(x.shape[0])(x, t, p_table)


# speedup vs baseline: 1.0023x; 1.0023x over previous
import functools

import jax
import jax.numpy as jnp
from jax import lax
from jax.experimental import pallas as pl
from jax.experimental.pallas import tpu as pltpu
from jax.experimental.pallas import tpu_sc as plsc

N_HID = 128
MAX_LEN = 240
LANES = 16
CHUNK = 128
NSLOT = 6


def _proj_table_body(emb_ref, w_ref, b_ref, out_ref):
    p = lax.dot_general(
        emb_ref[...], w_ref[...],
        dimension_numbers=(((1,), (1,)), ((), ())),
        preferred_element_type=jnp.float32,
    )
    out_ref[...] = p + b_ref[...]


@functools.cache
def _make_sc_kernel(n_rows):
    n_chunks = n_rows // CHUNK
    info = plsc.get_sparse_core_info()
    nc, ns = info.num_cores, info.num_subcores
    nw = nc * ns

    mesh = plsc.VectorSubcoreMesh(core_axis_name="c", subcore_axis_name="s")

    @functools.partial(
        pl.kernel,
        mesh=mesh,
        out_type=jax.ShapeDtypeStruct((n_rows, N_HID), jnp.float32),
        scratch_types=(
            [pltpu.VMEM((CHUNK,), jnp.int32) for _ in range(NSLOT)]
            + [pltpu.VMEM((CHUNK, N_HID), jnp.float32)
               for _ in range(NSLOT)]
            + [pltpu.VMEM_SHARED((MAX_LEN, N_HID), jnp.float32)]
            + [pltpu.SemaphoreType.DMA for _ in range(4 * NSLOT)]
        ),
    )
    def sc_fn(x_hbm, t_hbm, p_hbm, out_hbm, *scratch):
        iv = scratch[0:NSLOT]
        xv = scratch[NSLOT:2 * NSLOT]
        p_sh = scratch[2 * NSLOT]
        si = scratch[2 * NSLOT + 1:2 * NSLOT + 1 + NSLOT]
        sx = scratch[2 * NSLOT + 1 + NSLOT:2 * NSLOT + 1 + 2 * NSLOT]
        sg = scratch[2 * NSLOT + 1 + 2 * NSLOT:2 * NSLOT + 1 + 3 * NSLOT]
        so = scratch[2 * NSLOT + 1 + 3 * NSLOT:2 * NSLOT + 1 + 4 * NSLOT]

        wid = lax.axis_index("s") * nc + lax.axis_index("c")

        @pl.when(lax.axis_index("s") == 0)
        def _():
            pltpu.sync_copy(p_hbm, p_sh)
        plsc.subcore_barrier()

        def valid(m):
            return wid + m * nw < n_chunks

        def row_base(m):
            return (wid + m * nw) * CHUNK

        def issue_idx(m, s):
            @pl.when(valid(m))
            def _():
                pltpu.async_copy(
                    t_hbm.at[pl.ds(row_base(m), CHUNK)], iv[s], si[s])

        def issue_x(m, s):
            @pl.when(valid(m))
            def _():
                @pl.when(m >= NSLOT)
                def _():
                    pltpu.make_async_copy(
                        xv[s], out_hbm.at[pl.ds(0, CHUNK)], so[s]).wait()

                pltpu.async_copy(
                    x_hbm.at[pl.ds(row_base(m), CHUNK)], xv[s], sx[s])

        def issue_gadd(m, s):
            @pl.when(valid(m))
            def _():
                pltpu.make_async_copy(
                    t_hbm.at[pl.ds(row_base(m), CHUNK)], iv[s], si[s]).wait()
                pltpu.make_async_copy(
                    x_hbm.at[pl.ds(row_base(m), CHUNK)], xv[s], sx[s]).wait()
                pltpu.async_copy(p_sh.at[iv[s]], xv[s], sg[s], add=True)

        def finish(m, s):
            @pl.when(valid(m))
            def _():
                pltpu.make_async_copy(
                    p_sh.at[iv[s]], xv[s], sg[s]).wait()
                pltpu.async_copy(
                    xv[s], out_hbm.at[pl.ds(row_base(m), CHUNK)], so[s])

        for mm in range(NSLOT):
            issue_idx(mm, mm)
        issue_x(0, 0)
        issue_x(1, 1)
        issue_x(2, 2)
        issue_gadd(0, 0)

        def body(g, carry):
            for dm in range(NSLOT):
                m = g * NSLOT + dm
                issue_gadd(m + 1, (dm + 1) % NSLOT)
                finish(m, dm % NSLOT)
                issue_x(m + 3, (dm + 3) % NSLOT)
                issue_idx(m + NSLOT, dm % NSLOT)
            return carry

        per_w_max = -(-n_chunks // nw)
        lax.fori_loop(0, (per_w_max + NSLOT - 1) // NSLOT, body, 0)

        for s in range(NSLOT):
            pltpu.make_async_copy(
                xv[s], out_hbm.at[pl.ds(0, CHUNK)], so[s]).wait()

    return sc_fn


def kernel(x, t, emb_table, W, b):
    p_table = pl.pallas_call(
        _proj_table_body,
        out_shape=jax.ShapeDtypeStruct((MAX_LEN, N_HID), jnp.float32),
    )(emb_table, W, b.reshape(1, N_HID))
    return _make_sc_kernel(x.shape[0])(x, t, p_table)


# speedup vs baseline: 1.0098x; 1.0074x over previous
import functools

import jax
import jax.numpy as jnp
from jax import lax
from jax.experimental import pallas as pl
from jax.experimental.pallas import tpu as pltpu
from jax.experimental.pallas import tpu_sc as plsc

N_HID = 128
MAX_LEN = 240
LANES = 16
CHUNK = 128
NSLOT = 6


def _proj_table_body(emb_ref, w_ref, b_ref, out_ref):
    p = lax.dot_general(
        emb_ref[...], w_ref[...],
        dimension_numbers=(((1,), (1,)), ((), ())),
        preferred_element_type=jnp.float32,
    )
    out_ref[...] = p + b_ref[...]


@functools.cache
def _make_sc_kernel(n_rows):
    n_chunks = n_rows // CHUNK
    info = plsc.get_sparse_core_info()
    nc, ns = info.num_cores, info.num_subcores
    nw = nc * ns

    mesh = plsc.VectorSubcoreMesh(core_axis_name="c", subcore_axis_name="s")

    @functools.partial(
        pl.kernel,
        mesh=mesh,
        out_type=jax.ShapeDtypeStruct((n_rows, N_HID), jnp.float32),
        scratch_types=(
            [pltpu.VMEM((CHUNK,), jnp.int32) for _ in range(NSLOT)]
            + [pltpu.VMEM((CHUNK, N_HID), jnp.float32)
               for _ in range(NSLOT)]
            + [pltpu.VMEM_SHARED((MAX_LEN, N_HID), jnp.float32)]
            + [pltpu.SemaphoreType.DMA for _ in range(4 * NSLOT)]
        ),
    )
    def sc_fn(x_hbm, t_hbm, p_hbm, out_hbm, *scratch):
        iv = scratch[0:NSLOT]
        xv = scratch[NSLOT:2 * NSLOT]
        p_sh = scratch[2 * NSLOT]
        si = scratch[2 * NSLOT + 1:2 * NSLOT + 1 + NSLOT]
        sx = scratch[2 * NSLOT + 1 + NSLOT:2 * NSLOT + 1 + 2 * NSLOT]
        sg = scratch[2 * NSLOT + 1 + 2 * NSLOT:2 * NSLOT + 1 + 3 * NSLOT]
        so = scratch[2 * NSLOT + 1 + 3 * NSLOT:2 * NSLOT + 1 + 4 * NSLOT]

        wid = lax.axis_index("s") * nc + lax.axis_index("c")

        def valid(m):
            return wid + m * nw < n_chunks

        def row_base(m):
            return (wid + m * nw) * CHUNK

        def issue_idx(m, s):
            @pl.when(valid(m))
            def _():
                pltpu.async_copy(
                    t_hbm.at[pl.ds(row_base(m), CHUNK)], iv[s], si[s])

        def issue_x(m, s):
            @pl.when(valid(m))
            def _():
                @pl.when(m >= NSLOT)
                def _():
                    pltpu.make_async_copy(
                        xv[s], out_hbm.at[pl.ds(0, CHUNK)], so[s]).wait()

                pltpu.async_copy(
                    x_hbm.at[pl.ds(row_base(m), CHUNK)], xv[s], sx[s])

        def issue_gadd(m, s):
            @pl.when(valid(m))
            def _():
                pltpu.make_async_copy(
                    t_hbm.at[pl.ds(row_base(m), CHUNK)], iv[s], si[s]).wait()
                pltpu.make_async_copy(
                    x_hbm.at[pl.ds(row_base(m), CHUNK)], xv[s], sx[s]).wait()
                pltpu.async_copy(p_sh.at[iv[s]], xv[s], sg[s], add=True)

        def finish(m, s):
            @pl.when(valid(m))
            def _():
                pltpu.make_async_copy(
                    p_sh.at[iv[s]], xv[s], sg[s]).wait()
                pltpu.async_copy(
                    xv[s], out_hbm.at[pl.ds(row_base(m), CHUNK)], so[s])

        for mm in range(NSLOT):
            issue_idx(mm, mm)
        issue_x(0, 0)
        issue_x(1, 1)
        issue_x(2, 2)

        @pl.when(lax.axis_index("s") == 0)
        def _():
            pltpu.sync_copy(p_hbm, p_sh)
        plsc.subcore_barrier()

        issue_gadd(0, 0)

        def body(g, carry):
            for dm in range(NSLOT):
                m = g * NSLOT + dm
                issue_gadd(m + 1, (dm + 1) % NSLOT)
                finish(m, dm % NSLOT)
                issue_x(m + 3, (dm + 3) % NSLOT)
                issue_idx(m + NSLOT, dm % NSLOT)
            return carry

        per_w_max = -(-n_chunks // nw)
        lax.fori_loop(0, (per_w_max + NSLOT - 1) // NSLOT, body, 0)

        for s in range(NSLOT):
            pltpu.make_async_copy(
                xv[s], out_hbm.at[pl.ds(0, CHUNK)], so[s]).wait()

    return sc_fn


def kernel(x, t, emb_table, W, b):
    p_table = pl.pallas_call(
        _proj_table_body,
        out_shape=jax.ShapeDtypeStruct((MAX_LEN, N_HID), jnp.float32),
    )(emb_table, W, b.reshape(1, N_HID))
    return _make_sc_kernel(x.shape[0])(x, t, p_table)
